# Initial kernel scaffold; baseline (speedup 1.0000x reference)
import jax
import jax.numpy as jnp
from jax.experimental import pallas as pl


def kernel(x, edge_index, w_mean1, b_mean1, w_var1, b_var1, w_mean2, b_mean2, w_var2, b_var2):
    raise NotImplementedError("write your pallas kernel here")


# speedup vs baseline: 13.5076x; 13.5076x over previous
import functools

import numpy as np
import jax
import jax.numpy as jnp
from jax import lax
from jax.experimental import pallas as pl
from jax.experimental.pallas import tpu as pltpu
from jax.experimental.pallas import tpu_sc as plsc

_N = 10000
_E = 320000
_IN_F = 128
_HID = 16
_OUT_F = 64
_GAMMA = 1.0

_NC = 2
_NS = 16
_NW = _NC * _NS
_EPW = _E // _NW
_NP = 10240
_RPT = _NP // _NS


def _sc_mesh():
    return plsc.VectorSubcoreMesh(
        core_axis_name="c", subcore_axis_name="s", num_cores=_NC, num_subcores=_NS
    )


_SC_PARAMS = pltpu.CompilerParams(use_tc_tiling_on_sc=False)


def _degree_partials(dst):
    C = 2000
    n_chunks = _EPW // C

    @functools.partial(
        pl.kernel,
        out_type=jax.ShapeDtypeStruct((_NC, _NP, 8), jnp.float32),
        mesh=_sc_mesh(),
        scratch_types=[
            pltpu.VMEM((C,), jnp.int32),
            pltpu.VMEM((C, 8), jnp.float32),
            pltpu.VMEM_SHARED((_NP, 8), jnp.float32),
        ],
        compiler_params=_SC_PARAMS,
    )
    def body(dst_hbm, ones_hbm, zeros_hbm, out_hbm, idx_v, ones_v, acc_sp):
        c = lax.axis_index("c")
        s = lax.axis_index("s")
        wid = c * _NS + s
        r0 = s * _RPT
        pltpu.sync_copy(zeros_hbm.at[pl.ds(r0, _RPT)], acc_sp.at[pl.ds(r0, _RPT)])
        pltpu.sync_copy(ones_hbm, ones_v)
        plsc.subcore_barrier()
        base = wid * _EPW
        for k in range(n_chunks):
            pltpu.sync_copy(dst_hbm.at[pl.ds(base + k * C, C)], idx_v)
            pltpu.sync_copy(ones_v, acc_sp.at[idx_v], add=True)
        plsc.subcore_barrier()
        pltpu.sync_copy(acc_sp.at[pl.ds(r0, _RPT)], out_hbm.at[c, pl.ds(r0, _RPT)])

    return body(dst, jnp.ones((C, 8), jnp.float32), jnp.zeros((_NP, 8), jnp.float32))


def _propagate(h, src, dst, feat, chunk):
    n_chunks = _EPW // chunk

    @functools.partial(
        pl.kernel,
        out_type=jax.ShapeDtypeStruct((_NC, _NP, feat), jnp.float32),
        mesh=_sc_mesh(),
        scratch_types=[
            pltpu.VMEM((chunk,), jnp.int32),
            pltpu.VMEM((chunk,), jnp.int32),
            pltpu.VMEM((chunk, feat), jnp.float32),
            pltpu.VMEM_SHARED((_NP, feat), jnp.float32),
            pltpu.SemaphoreType.DMA,
        ],
        compiler_params=_SC_PARAMS,
    )
    def body(h_hbm, src_hbm, dst_hbm, zeros_hbm, out_hbm, si_v, di_v, rows_v, acc_sp, sem):
        c = lax.axis_index("c")
        s = lax.axis_index("s")
        wid = c * _NS + s
        r0 = s * _RPT
        pltpu.sync_copy(zeros_hbm.at[pl.ds(r0, _RPT)], acc_sp.at[pl.ds(r0, _RPT)])
        plsc.subcore_barrier()
        base = wid * _EPW
        for k in range(n_chunks):
            off = base + k * chunk
            pltpu.sync_copy(src_hbm.at[pl.ds(off, chunk)], si_v)
            pltpu.sync_copy(dst_hbm.at[pl.ds(off, chunk)], di_v)
            pltpu.async_copy(h_hbm.at[si_v], rows_v, sem).wait()
            pltpu.sync_copy(rows_v, acc_sp.at[di_v], add=True)
        plsc.subcore_barrier()
        pltpu.sync_copy(acc_sp.at[pl.ds(r0, _RPT)], out_hbm.at[c, pl.ds(r0, _RPT)])

    return body(h, src, dst, jnp.zeros((_NP, feat), jnp.float32))


def _propagate_split(h2flat, srcs, dst):
    feat = _OUT_F
    chunk = 1000
    ept = _E // _NS
    n_chunks = ept // chunk

    @functools.partial(
        pl.kernel,
        out_type=jax.ShapeDtypeStruct((_NC, _NP, feat), jnp.float32),
        mesh=_sc_mesh(),
        scratch_types=[
            pltpu.VMEM((chunk,), jnp.int32),
            pltpu.VMEM((chunk,), jnp.int32),
            pltpu.VMEM((chunk, feat), jnp.float32),
            pltpu.VMEM_SHARED((_NP, feat), jnp.float32),
            pltpu.SemaphoreType.DMA,
        ],
        compiler_params=_SC_PARAMS,
    )
    def body(h_hbm, srcs_hbm, dst_hbm, zeros_hbm, out_hbm, si_v, di_v, rows_v, acc_sp, sem):
        c = lax.axis_index("c")
        s = lax.axis_index("s")
        r0 = s * _RPT
        pltpu.sync_copy(zeros_hbm.at[pl.ds(r0, _RPT)], acc_sp.at[pl.ds(r0, _RPT)])
        plsc.subcore_barrier()
        base = s * ept
        for k in range(n_chunks):
            off = base + k * chunk
            pltpu.sync_copy(srcs_hbm.at[c, pl.ds(off, chunk)], si_v)
            pltpu.sync_copy(dst_hbm.at[pl.ds(off, chunk)], di_v)
            pltpu.async_copy(h_hbm.at[si_v], rows_v, sem).wait()
            pltpu.sync_copy(rows_v, acc_sp.at[di_v], add=True)
        plsc.subcore_barrier()
        pltpu.sync_copy(acc_sp.at[pl.ds(r0, _RPT)], out_hbm.at[c, pl.ds(r0, _RPT)])

    return body(h2flat, srcs, dst, jnp.zeros((_NP, feat), jnp.float32))


_R = 1000


def _layer1_dense(x, wm, bm, wv, bv, degp):
    def body(x_ref, wm_ref, bm_ref, wv_ref, bv_ref, dp_ref, out_ref):
        xb = x_ref[...]
        mean = jnp.dot(xb, wm_ref[...], preferred_element_type=jnp.float32) + bm_ref[...]
        var = jnp.dot(xb, wv_ref[...], preferred_element_type=jnp.float32) + bv_ref[...]
        mean = jnp.maximum(mean, 0.0)
        var = jnp.maximum(var, 0.0)
        att = jnp.exp(-_GAMMA * var)
        mean = mean * att
        var = var * att * att
        deg = jnp.maximum(dp_ref[0, :, :1] + dp_ref[1, :, :1], 1.0)
        n1 = lax.rsqrt(deg)
        n2 = 1.0 / deg
        out_ref[...] = jnp.concatenate([mean * n1, var * n2], axis=1)

    return pl.pallas_call(
        body,
        grid=(_N // _R,),
        in_specs=[
            pl.BlockSpec((_R, _IN_F), lambda i: (i, 0)),
            pl.BlockSpec((_IN_F, _HID), lambda i: (0, 0)),
            pl.BlockSpec((1, _HID), lambda i: (0, 0)),
            pl.BlockSpec((_IN_F, _HID), lambda i: (0, 0)),
            pl.BlockSpec((1, _HID), lambda i: (0, 0)),
            pl.BlockSpec((_NC, _R, 8), lambda i: (0, i, 0)),
        ],
        out_specs=pl.BlockSpec((_R, 2 * _HID), lambda i: (i, 0)),
        out_shape=jax.ShapeDtypeStruct((_N, 2 * _HID), jnp.float32),
    )(x, wm, bm.reshape(1, -1), wv, bv.reshape(1, -1), degp)


def _layer2_dense(g1, degp, wm, bm, wv, bv):
    def body(g_ref, dp_ref, wm_ref, bm_ref, wv_ref, bv_ref, out_ref):
        deg = jnp.maximum(dp_ref[0, :, :1] + dp_ref[1, :, :1], 1.0)
        n1 = lax.rsqrt(deg)
        n2 = 1.0 / deg
        sblk = g_ref[0] + g_ref[1]
        m1 = sblk[:, :_HID] * n1
        v1 = sblk[:, _HID:] * n2
        m2 = jnp.dot(m1, wm_ref[...], preferred_element_type=jnp.float32) + bm_ref[...]
        v2 = jnp.dot(v1, wv_ref[...], preferred_element_type=jnp.float32) + bv_ref[...]
        v2 = jnp.maximum(v2, 0.0)
        att = jnp.exp(-_GAMMA * v2)
        m2 = m2 * att
        v2 = v2 * att * att
        out_ref[0] = m2 * n1
        out_ref[1] = v2 * n2

    return pl.pallas_call(
        body,
        grid=(_N // _R,),
        in_specs=[
            pl.BlockSpec((_NC, _R, 2 * _HID), lambda i: (0, i, 0)),
            pl.BlockSpec((_NC, _R, 8), lambda i: (0, i, 0)),
            pl.BlockSpec((_HID, _OUT_F), lambda i: (0, 0)),
            pl.BlockSpec((1, _OUT_F), lambda i: (0, 0)),
            pl.BlockSpec((_HID, _OUT_F), lambda i: (0, 0)),
            pl.BlockSpec((1, _OUT_F), lambda i: (0, 0)),
        ],
        out_specs=pl.BlockSpec((2, _R, _OUT_F), lambda i: (0, i, 0)),
        out_shape=jax.ShapeDtypeStruct((2, _NP, _OUT_F), jnp.float32),
    )(g1, degp, wm, bm.reshape(1, -1), wv, bv.reshape(1, -1))


def _finalize(g2, degp, eps):
    def body(g_ref, dp_ref, eps_ref, out_ref):
        deg = jnp.maximum(dp_ref[0, :, :1] + dp_ref[1, :, :1], 1.0)
        n1 = lax.rsqrt(deg)
        n2 = 1.0 / deg
        mean = g_ref[0] * n1
        var = g_ref[1] * n2
        out_ref[...] = eps_ref[...] * jnp.sqrt(var + 1e-8) + mean

    return pl.pallas_call(
        body,
        grid=(_N // _R,),
        in_specs=[
            pl.BlockSpec((2, _R, _OUT_F), lambda i: (0, i, 0)),
            pl.BlockSpec((_NC, _R, 8), lambda i: (0, i, 0)),
            pl.BlockSpec((_R, _OUT_F), lambda i: (i, 0)),
        ],
        out_specs=pl.BlockSpec((_R, _OUT_F), lambda i: (i, 0)),
        out_shape=jax.ShapeDtypeStruct((_N, _OUT_F), jnp.float32),
    )(g2, degp, eps)


def kernel(x, edge_index, w_mean1, b_mean1, w_var1, b_var1, w_mean2, b_mean2, w_var2, b_var2):
    src = edge_index[0]
    dst = edge_index[1]
    degp = _degree_partials(dst)
    h1 = _layer1_dense(x, w_mean1, b_mean1, w_var1, b_var1, degp)
    g1 = _propagate(h1, src, dst, 2 * _HID, 2000)
    h2s = _layer2_dense(g1, degp, w_mean2, b_mean2, w_var2, b_var2)
    h2flat = h2s.reshape(2 * _NP, _OUT_F)
    srcs = jnp.stack([src, src + _NP])
    g2 = _propagate_split(h2flat, srcs, dst)
    eps = jax.random.normal(jax.random.key(42), (_N, _OUT_F), dtype=jnp.float32)
    return _finalize(g2, degp, eps)


# speedup vs baseline: 13.7966x; 1.0214x over previous
import functools

import numpy as np
import jax
import jax.numpy as jnp
from jax import lax
from jax.experimental import pallas as pl
from jax.experimental.pallas import tpu as pltpu
from jax.experimental.pallas import tpu_sc as plsc

_N = 10000
_E = 320000
_IN_F = 128
_HID = 16
_OUT_F = 64
_GAMMA = 1.0

_NC = 2
_NS = 16
_NW = _NC * _NS
_EPW = _E // _NW
_NP = 10240
_RPT = _NP // _NS


def _sc_mesh():
    return plsc.VectorSubcoreMesh(
        core_axis_name="c", subcore_axis_name="s", num_cores=_NC, num_subcores=_NS
    )


_SC_PARAMS = pltpu.CompilerParams(use_tc_tiling_on_sc=False)


def _degree_partials(dst):
    C = 2000
    n_chunks = _EPW // C

    @functools.partial(
        pl.kernel,
        out_type=jax.ShapeDtypeStruct((_NC, _NP, 8), jnp.float32),
        mesh=_sc_mesh(),
        scratch_types=[
            pltpu.VMEM((C,), jnp.int32),
            pltpu.VMEM((C, 8), jnp.float32),
            pltpu.VMEM_SHARED((_NP, 8), jnp.float32),
        ],
        compiler_params=_SC_PARAMS,
    )
    def body(dst_hbm, ones_hbm, zeros_hbm, out_hbm, idx_v, ones_v, acc_sp):
        c = lax.axis_index("c")
        s = lax.axis_index("s")
        wid = c * _NS + s
        r0 = s * _RPT
        pltpu.sync_copy(zeros_hbm.at[pl.ds(r0, _RPT)], acc_sp.at[pl.ds(r0, _RPT)])
        pltpu.sync_copy(ones_hbm, ones_v)
        plsc.subcore_barrier()
        base = wid * _EPW
        for k in range(n_chunks):
            pltpu.sync_copy(dst_hbm.at[pl.ds(base + k * C, C)], idx_v)
            pltpu.sync_copy(ones_v, acc_sp.at[idx_v], add=True)
        plsc.subcore_barrier()
        pltpu.sync_copy(acc_sp.at[pl.ds(r0, _RPT)], out_hbm.at[c, pl.ds(r0, _RPT)])

    return body(dst, jnp.ones((C, 8), jnp.float32), jnp.zeros((_NP, 8), jnp.float32))


def _propagate_pipelined(h, srcs, dst, feat, chunk, split):
    ept = _E // _NS if split else _EPW
    n_chunks = ept // chunk
    assert ept % chunk == 0 and chunk % 8 == 0

    @functools.partial(
        pl.kernel,
        out_type=jax.ShapeDtypeStruct((_NC, _NP, feat), jnp.float32),
        mesh=_sc_mesh(),
        scratch_types=[
            [pltpu.VMEM((chunk,), jnp.int32)] * 2,
            [pltpu.VMEM((chunk,), jnp.int32)] * 2,
            [pltpu.VMEM((chunk, feat), jnp.float32)] * 2,
            pltpu.VMEM_SHARED((_NP, feat), jnp.float32),
            [pltpu.SemaphoreType.DMA] * 2,
            [pltpu.SemaphoreType.DMA] * 2,
        ],
        compiler_params=_SC_PARAMS,
    )
    def body(h_hbm, srcs_hbm, dst_hbm, zeros_hbm, out_hbm, si, di, rows, acc_sp, gsem, ssem):
        c = lax.axis_index("c")
        s = lax.axis_index("s")
        r0 = s * _RPT
        pltpu.sync_copy(zeros_hbm.at[pl.ds(r0, _RPT)], acc_sp.at[pl.ds(r0, _RPT)])
        plsc.subcore_barrier()
        base = (s * ept) if split else ((c * _NS + s) * ept)

        def load_idx(k, b):
            off = base + k * chunk
            if split:
                pltpu.sync_copy(srcs_hbm.at[c, pl.ds(off, chunk)], si[b])
            else:
                pltpu.sync_copy(srcs_hbm.at[pl.ds(off, chunk)], si[b])
            pltpu.sync_copy(dst_hbm.at[pl.ds(off, chunk)], di[b])

        load_idx(0, 0)
        g0 = pltpu.async_copy(h_hbm.at[si[0]], rows[0], gsem[0])
        scat_prev = None
        for k in range(n_chunks):
            b = k % 2
            (g0 if k == 0 else gk).wait()
            sk = pltpu.async_copy(rows[b], acc_sp.at[di[b]], ssem[b], add=True)
            if k + 1 < n_chunks:
                if scat_prev is not None:
                    scat_prev.wait()
                load_idx(k + 1, 1 - b)
                gk = pltpu.async_copy(h_hbm.at[si[1 - b]], rows[1 - b], gsem[1 - b])
            scat_prev = sk
        scat_prev.wait()
        plsc.subcore_barrier()
        pltpu.sync_copy(acc_sp.at[pl.ds(r0, _RPT)], out_hbm.at[c, pl.ds(r0, _RPT)])

    return body(h, srcs, dst, jnp.zeros((_NP, feat), jnp.float32))


_R = 1000


def _layer1_dense(x, wm, bm, wv, bv, degp):
    def body(x_ref, wm_ref, bm_ref, wv_ref, bv_ref, dp_ref, out_ref):
        xb = x_ref[...]
        mean = jnp.dot(xb, wm_ref[...], preferred_element_type=jnp.float32) + bm_ref[...]
        var = jnp.dot(xb, wv_ref[...], preferred_element_type=jnp.float32) + bv_ref[...]
        mean = jnp.maximum(mean, 0.0)
        var = jnp.maximum(var, 0.0)
        att = jnp.exp(-_GAMMA * var)
        mean = mean * att
        var = var * att * att
        deg = jnp.maximum(dp_ref[0, :, :1] + dp_ref[1, :, :1], 1.0)
        n1 = lax.rsqrt(deg)
        n2 = 1.0 / deg
        out_ref[...] = jnp.concatenate([mean * n1, var * n2], axis=1)

    return pl.pallas_call(
        body,
        grid=(_N // _R,),
        in_specs=[
            pl.BlockSpec((_R, _IN_F), lambda i: (i, 0)),
            pl.BlockSpec((_IN_F, _HID), lambda i: (0, 0)),
            pl.BlockSpec((1, _HID), lambda i: (0, 0)),
            pl.BlockSpec((_IN_F, _HID), lambda i: (0, 0)),
            pl.BlockSpec((1, _HID), lambda i: (0, 0)),
            pl.BlockSpec((_NC, _R, 8), lambda i: (0, i, 0)),
        ],
        out_specs=pl.BlockSpec((_R, 2 * _HID), lambda i: (i, 0)),
        out_shape=jax.ShapeDtypeStruct((_N, 2 * _HID), jnp.float32),
    )(x, wm, bm.reshape(1, -1), wv, bv.reshape(1, -1), degp)


def _layer2_dense(g1, degp, wm, bm, wv, bv):
    def body(g_ref, dp_ref, wm_ref, bm_ref, wv_ref, bv_ref, out_ref):
        deg = jnp.maximum(dp_ref[0, :, :1] + dp_ref[1, :, :1], 1.0)
        n1 = lax.rsqrt(deg)
        n2 = 1.0 / deg
        sblk = g_ref[0] + g_ref[1]
        m1 = sblk[:, :_HID] * n1
        v1 = sblk[:, _HID:] * n2
        m2 = jnp.dot(m1, wm_ref[...], preferred_element_type=jnp.float32) + bm_ref[...]
        v2 = jnp.dot(v1, wv_ref[...], preferred_element_type=jnp.float32) + bv_ref[...]
        v2 = jnp.maximum(v2, 0.0)
        att = jnp.exp(-_GAMMA * v2)
        m2 = m2 * att
        v2 = v2 * att * att
        out_ref[0] = m2 * n1
        out_ref[1] = v2 * n2

    return pl.pallas_call(
        body,
        grid=(_N // _R,),
        in_specs=[
            pl.BlockSpec((_NC, _R, 2 * _HID), lambda i: (0, i, 0)),
            pl.BlockSpec((_NC, _R, 8), lambda i: (0, i, 0)),
            pl.BlockSpec((_HID, _OUT_F), lambda i: (0, 0)),
            pl.BlockSpec((1, _OUT_F), lambda i: (0, 0)),
            pl.BlockSpec((_HID, _OUT_F), lambda i: (0, 0)),
            pl.BlockSpec((1, _OUT_F), lambda i: (0, 0)),
        ],
        out_specs=pl.BlockSpec((2, _R, _OUT_F), lambda i: (0, i, 0)),
        out_shape=jax.ShapeDtypeStruct((2, _NP, _OUT_F), jnp.float32),
    )(g1, degp, wm, bm.reshape(1, -1), wv, bv.reshape(1, -1))


def _finalize(g2, degp, eps):
    def body(g_ref, dp_ref, eps_ref, out_ref):
        deg = jnp.maximum(dp_ref[0, :, :1] + dp_ref[1, :, :1], 1.0)
        n1 = lax.rsqrt(deg)
        n2 = 1.0 / deg
        mean = g_ref[0] * n1
        var = g_ref[1] * n2
        out_ref[...] = eps_ref[...] * jnp.sqrt(var + 1e-8) + mean

    return pl.pallas_call(
        body,
        grid=(_N // _R,),
        in_specs=[
            pl.BlockSpec((2, _R, _OUT_F), lambda i: (0, i, 0)),
            pl.BlockSpec((_NC, _R, 8), lambda i: (0, i, 0)),
            pl.BlockSpec((_R, _OUT_F), lambda i: (i, 0)),
        ],
        out_specs=pl.BlockSpec((_R, _OUT_F), lambda i: (i, 0)),
        out_shape=jax.ShapeDtypeStruct((_N, _OUT_F), jnp.float32),
    )(g2, degp, eps)


def kernel(x, edge_index, w_mean1, b_mean1, w_var1, b_var1, w_mean2, b_mean2, w_var2, b_var2):
    src = edge_index[0]
    dst = edge_index[1]
    degp = _degree_partials(dst)
    h1 = _layer1_dense(x, w_mean1, b_mean1, w_var1, b_var1, degp)
    g1 = _propagate_pipelined(h1, src, dst, 2 * _HID, 1000, split=False)
    h2s = _layer2_dense(g1, degp, w_mean2, b_mean2, w_var2, b_var2)
    h2flat = h2s.reshape(2 * _NP, _OUT_F)
    srcs = jnp.stack([src, src + _NP])
    g2 = _propagate_pipelined(h2flat, srcs, dst, _OUT_F, 400, split=True)
    eps = jax.random.normal(jax.random.key(42), (_N, _OUT_F), dtype=jnp.float32)
    return _finalize(g2, degp, eps)


# speedup vs baseline: 14.0145x; 1.0158x over previous
import functools

import numpy as np
import jax
import jax.numpy as jnp
from jax import lax
from jax.experimental import pallas as pl
from jax.experimental.pallas import tpu as pltpu
from jax.experimental.pallas import tpu_sc as plsc

_N = 10000
_E = 320000
_IN_F = 128
_HID = 16
_OUT_F = 64
_GAMMA = 1.0

_NC = 2
_NS = 16
_NW = _NC * _NS
_EPW = _E // _NW
_NP = 10240
_RPT = _NP // _NS


def _sc_mesh():
    return plsc.VectorSubcoreMesh(
        core_axis_name="c", subcore_axis_name="s", num_cores=_NC, num_subcores=_NS
    )


_SC_PARAMS = pltpu.CompilerParams(use_tc_tiling_on_sc=False)


def _degree_partials(dst):
    C = 2000
    n_chunks = _EPW // C

    @functools.partial(
        pl.kernel,
        out_type=jax.ShapeDtypeStruct((_NC, _NP, 8), jnp.float32),
        mesh=_sc_mesh(),
        scratch_types=[
            pltpu.VMEM((C,), jnp.int32),
            pltpu.VMEM((C, 8), jnp.float32),
            pltpu.VMEM_SHARED((_NP, 8), jnp.float32),
        ],
        compiler_params=_SC_PARAMS,
    )
    def body(dst_hbm, ones_hbm, zeros_hbm, out_hbm, idx_v, ones_v, acc_sp):
        c = lax.axis_index("c")
        s = lax.axis_index("s")
        wid = c * _NS + s
        r0 = s * _RPT
        pltpu.sync_copy(zeros_hbm.at[pl.ds(r0, _RPT)], acc_sp.at[pl.ds(r0, _RPT)])
        pltpu.sync_copy(ones_hbm, ones_v)
        plsc.subcore_barrier()
        base = wid * _EPW
        for k in range(n_chunks):
            pltpu.sync_copy(dst_hbm.at[pl.ds(base + k * C, C)], idx_v)
            pltpu.sync_copy(ones_v, acc_sp.at[idx_v], add=True)
        plsc.subcore_barrier()
        pltpu.sync_copy(acc_sp.at[pl.ds(r0, _RPT)], out_hbm.at[c, pl.ds(r0, _RPT)])

    return body(dst, jnp.ones((C, 8), jnp.float32), jnp.zeros((_NP, 8), jnp.float32))


def _propagate_pipelined(h, srcs, dst, feat, chunk, split):
    ept = _E // _NS if split else _EPW
    n_chunks = ept // chunk
    assert ept % chunk == 0 and chunk % 8 == 0

    @functools.partial(
        pl.kernel,
        out_type=jax.ShapeDtypeStruct((_NC, _NP, feat), jnp.float32),
        mesh=_sc_mesh(),
        scratch_types=[
            [pltpu.VMEM((chunk,), jnp.int32)] * 2,
            [pltpu.VMEM((chunk,), jnp.int32)] * 2,
            [pltpu.VMEM((chunk, feat), jnp.float32)] * 2,
            pltpu.VMEM_SHARED((_NP, feat), jnp.float32),
            [pltpu.SemaphoreType.DMA] * 2,
            [pltpu.SemaphoreType.DMA] * 2,
        ],
        compiler_params=_SC_PARAMS,
    )
    def body(h_hbm, srcs_hbm, dst_hbm, out_hbm, si, di, rows, acc_sp, gsem, ssem):
        c = lax.axis_index("c")
        s = lax.axis_index("s")
        r0 = s * _RPT
        zrows = min(chunk, _RPT)
        zero = jnp.zeros((16,), jnp.float32)

        def zrow(i, carry):
            for j in range(feat // 16):
                rows[0][i, pl.ds(j * 16, 16)] = zero
            return carry

        lax.fori_loop(0, zrows, zrow, 0)
        off0 = 0
        while off0 < _RPT:
            n = min(zrows, _RPT - off0)
            pltpu.sync_copy(rows[0].at[pl.ds(0, n)], acc_sp.at[pl.ds(r0 + off0, n)])
            off0 += n
        plsc.subcore_barrier()
        base = (s * ept) if split else ((c * _NS + s) * ept)

        def load_idx(k, b):
            off = base + k * chunk
            if split:
                pltpu.sync_copy(srcs_hbm.at[c, pl.ds(off, chunk)], si[b])
            else:
                pltpu.sync_copy(srcs_hbm.at[pl.ds(off, chunk)], si[b])
            pltpu.sync_copy(dst_hbm.at[pl.ds(off, chunk)], di[b])

        load_idx(0, 0)
        g0 = pltpu.async_copy(h_hbm.at[si[0]], rows[0], gsem[0])
        scat_prev = None
        for k in range(n_chunks):
            b = k % 2
            (g0 if k == 0 else gk).wait()
            sk = pltpu.async_copy(rows[b], acc_sp.at[di[b]], ssem[b], add=True)
            if k + 1 < n_chunks:
                if scat_prev is not None:
                    scat_prev.wait()
                load_idx(k + 1, 1 - b)
                gk = pltpu.async_copy(h_hbm.at[si[1 - b]], rows[1 - b], gsem[1 - b])
            scat_prev = sk
        scat_prev.wait()
        plsc.subcore_barrier()
        pltpu.sync_copy(acc_sp.at[pl.ds(r0, _RPT)], out_hbm.at[c, pl.ds(r0, _RPT)])

    return body(h, srcs, dst)


_R = 1000


def _layer1_matmul(x, wm, bm, wv, bv):
    def body(x_ref, wm_ref, bm_ref, wv_ref, bv_ref, out_ref):
        xb = x_ref[...]
        mean = jnp.dot(xb, wm_ref[...], preferred_element_type=jnp.float32) + bm_ref[...]
        var = jnp.dot(xb, wv_ref[...], preferred_element_type=jnp.float32) + bv_ref[...]
        mean = jnp.maximum(mean, 0.0)
        var = jnp.maximum(var, 0.0)
        att = jnp.exp(-_GAMMA * var)
        out_ref[...] = jnp.concatenate([mean * att, var * att * att], axis=1)

    return pl.pallas_call(
        body,
        grid=(_N // _R,),
        in_specs=[
            pl.BlockSpec((_R, _IN_F), lambda i: (i, 0)),
            pl.BlockSpec((_IN_F, _HID), lambda i: (0, 0)),
            pl.BlockSpec((1, _HID), lambda i: (0, 0)),
            pl.BlockSpec((_IN_F, _HID), lambda i: (0, 0)),
            pl.BlockSpec((1, _HID), lambda i: (0, 0)),
        ],
        out_specs=pl.BlockSpec((_R, 2 * _HID), lambda i: (i, 0)),
        out_shape=jax.ShapeDtypeStruct((_N, 2 * _HID), jnp.float32),
    )(x, wm, bm.reshape(1, -1), wv, bv.reshape(1, -1))


def _scale1(h1u, degp):
    def body(h_ref, dp_ref, out_ref):
        deg = jnp.maximum(dp_ref[0, :, :1] + dp_ref[1, :, :1], 1.0)
        n1 = lax.rsqrt(deg)
        n2 = 1.0 / deg
        hb = h_ref[...]
        out_ref[...] = jnp.concatenate(
            [hb[:, :_HID] * n1, hb[:, _HID:] * n2], axis=1)

    return pl.pallas_call(
        body,
        grid=(_N // _R,),
        in_specs=[
            pl.BlockSpec((_R, 2 * _HID), lambda i: (i, 0)),
            pl.BlockSpec((_NC, _R, 8), lambda i: (0, i, 0)),
        ],
        out_specs=pl.BlockSpec((_R, 2 * _HID), lambda i: (i, 0)),
        out_shape=jax.ShapeDtypeStruct((_N, 2 * _HID), jnp.float32),
    )(h1u, degp)


def _layer2_dense(g1, degp, wm, bm, wv, bv):
    def body(g_ref, dp_ref, wm_ref, bm_ref, wv_ref, bv_ref, out_ref):
        deg = jnp.maximum(dp_ref[0, :, :1] + dp_ref[1, :, :1], 1.0)
        n1 = lax.rsqrt(deg)
        n2 = 1.0 / deg
        sblk = g_ref[0] + g_ref[1]
        m1 = sblk[:, :_HID] * n1
        v1 = sblk[:, _HID:] * n2
        m2 = jnp.dot(m1, wm_ref[...], preferred_element_type=jnp.float32) + bm_ref[...]
        v2 = jnp.dot(v1, wv_ref[...], preferred_element_type=jnp.float32) + bv_ref[...]
        v2 = jnp.maximum(v2, 0.0)
        att = jnp.exp(-_GAMMA * v2)
        m2 = m2 * att
        v2 = v2 * att * att
        out_ref[0] = m2 * n1
        out_ref[1] = v2 * n2

    return pl.pallas_call(
        body,
        grid=(_N // _R,),
        in_specs=[
            pl.BlockSpec((_NC, _R, 2 * _HID), lambda i: (0, i, 0)),
            pl.BlockSpec((_NC, _R, 8), lambda i: (0, i, 0)),
            pl.BlockSpec((_HID, _OUT_F), lambda i: (0, 0)),
            pl.BlockSpec((1, _OUT_F), lambda i: (0, 0)),
            pl.BlockSpec((_HID, _OUT_F), lambda i: (0, 0)),
            pl.BlockSpec((1, _OUT_F), lambda i: (0, 0)),
        ],
        out_specs=pl.BlockSpec((2, _R, _OUT_F), lambda i: (0, i, 0)),
        out_shape=jax.ShapeDtypeStruct((2, _NP, _OUT_F), jnp.float32),
    )(g1, degp, wm, bm.reshape(1, -1), wv, bv.reshape(1, -1))


def _finalize(g2, degp, eps):
    def body(g_ref, dp_ref, eps_ref, out_ref):
        deg = jnp.maximum(dp_ref[0, :, :1] + dp_ref[1, :, :1], 1.0)
        n1 = lax.rsqrt(deg)
        n2 = 1.0 / deg
        mean = g_ref[0] * n1
        var = g_ref[1] * n2
        out_ref[...] = eps_ref[...] * jnp.sqrt(var + 1e-8) + mean

    return pl.pallas_call(
        body,
        grid=(_N // _R,),
        in_specs=[
            pl.BlockSpec((2, _R, _OUT_F), lambda i: (0, i, 0)),
            pl.BlockSpec((_NC, _R, 8), lambda i: (0, i, 0)),
            pl.BlockSpec((_R, _OUT_F), lambda i: (i, 0)),
        ],
        out_specs=pl.BlockSpec((_R, _OUT_F), lambda i: (i, 0)),
        out_shape=jax.ShapeDtypeStruct((_N, _OUT_F), jnp.float32),
    )(g2, degp, eps)


def kernel(x, edge_index, w_mean1, b_mean1, w_var1, b_var1, w_mean2, b_mean2, w_var2, b_var2):
    src = edge_index[0]
    dst = edge_index[1]
    degp = _degree_partials(dst)
    h1u = _layer1_matmul(x, w_mean1, b_mean1, w_var1, b_var1)
    h1 = _scale1(h1u, degp)
    g1 = _propagate_pipelined(h1, src, dst, 2 * _HID, 1000, split=False)
    h2s = _layer2_dense(g1, degp, w_mean2, b_mean2, w_var2, b_var2)
    h2flat = h2s.reshape(2 * _NP, _OUT_F)
    srcs = jnp.stack([src, src + _NP])
    g2 = _propagate_pipelined(h2flat, srcs, dst, _OUT_F, 400, split=True)
    eps = jax.random.normal(jax.random.key(42), (_N, _OUT_F), dtype=jnp.float32)
    return _finalize(g2, degp, eps)
